# baseline (speedup 1.0000x reference)
import functools

import jax
import jax.numpy as jnp
from jax import lax
from jax.experimental import pallas as pl
from jax.experimental.pallas import tpu as pltpu
from jax.experimental.pallas import tpu_sc as plsc

N = 50000
NP = 50176
E = 800000
EP = 802816
HID = 128
FC = 16
FH = 8
BN = 512
STRIPE = NP // 16


def _fill(ref, rows, w, val):
    v = jnp.full((16,), val, ref.dtype)

    def body(i, carry):
        for k in range(w // 16):
            ref[i, k * 16:(k + 1) * 16] = v
        return carry

    lax.fori_loop(0, rows, body, 0)


def _zero_stripe(zbuf, acc_sh, base):
    def body(k, carry):
        pltpu.sync_copy(zbuf, acc_sh.at[pl.ds(base + k * 128, 128)])
        return carry

    lax.fori_loop(0, STRIPE // 128, body, 0)
    rem = STRIPE % 128
    if rem:
        pltpu.sync_copy(zbuf.at[pl.ds(0, rem)],
                        acc_sh.at[pl.ds(base + (STRIPE // 128) * 128, rem)])


def _agg_chunk(g2, out, src_v, dst_v, rows0, rows1, sem0, sem1, zbuf,
               acc_sh, base):
    _zero_stripe(zbuf, acc_sh, base)
    plsc.subcore_barrier()

    def body(u, carry):
        j0 = 2 * u
        j1 = 2 * u + 1
        pltpu.async_copy(g2.at[src_v.at[j0]], rows0, sem0)
        pltpu.async_copy(g2.at[src_v.at[j1]], rows1, sem1)
        pltpu.make_async_copy(g2.at[src_v.at[j0]], rows0, sem0).wait()
        pltpu.sync_copy(rows0, acc_sh.at[dst_v.at[j0]], add=True)
        pltpu.make_async_copy(g2.at[src_v.at[j1]], rows1, sem1).wait()
        pltpu.sync_copy(rows1, acc_sh.at[dst_v.at[j1]], add=True)
        return carry

    lax.fori_loop(0, 196, body, 0)
    plsc.subcore_barrier()
    pltpu.sync_copy(acc_sh.at[pl.ds(base, STRIPE)], out.at[pl.ds(base, STRIPE)])
    plsc.subcore_barrier()


def _bump(src_v, delta):
    v = jnp.full((16,), delta, jnp.int32)

    def body(j, carry):
        for k in range(8):
            src_v[j, k * 16:(k + 1) * 16] = src_v[j, k * 16:(k + 1) * 16] + v
        return carry

    lax.fori_loop(0, 392, body, 0)


@functools.cache
def _build_sc_kernels():
    mesh = plsc.VectorSubcoreMesh(core_axis_name="c", subcore_axis_name="s")

    @functools.partial(
        pl.kernel,
        mesh=mesh,
        compiler_params=pltpu.CompilerParams(use_tc_tiling_on_sc=False),
        out_type=jax.ShapeDtypeStruct((2 * NP, FH), jnp.float32),
        scratch_types=[
            pltpu.VMEM((196, 128), jnp.int32),
            pltpu.VMEM((128, FH), jnp.float32),
            pltpu.VMEM((128, FH), jnp.float32),
            pltpu.VMEM_SHARED((NP, FH), jnp.float32),
        ],
    )
    def _deg_kernel(dst_hbm, out_hbm, idx_v, ones_v, zbuf, acc_sh):
        c = lax.axis_index("c")
        s = lax.axis_index("s")
        w = c * 16 + s
        base = s * STRIPE
        _fill(ones_v, 128, FH, 1.0)
        _fill(zbuf, 128, FH, 0.0)
        _zero_stripe(zbuf, acc_sh, base)
        pltpu.sync_copy(dst_hbm.at[w], idx_v)
        plsc.subcore_barrier()

        def body(j, carry):
            pltpu.sync_copy(ones_v, acc_sh.at[idx_v.at[j]], add=True)
            return carry

        lax.fori_loop(0, 196, body, 0)
        plsc.subcore_barrier()
        pltpu.sync_copy(acc_sh.at[pl.ds(base, STRIPE)],
                        out_hbm.at[pl.ds(c * NP + base, STRIPE)])

    @functools.partial(
        pl.kernel,
        mesh=mesh,
        compiler_params=pltpu.CompilerParams(
            use_tc_tiling_on_sc=False,
            disable_bounds_checks=True,
            disable_semaphore_checks=True),
        out_type=tuple(jax.ShapeDtypeStruct((NP, FH), jnp.float32)
                       for _ in range(16)),
        scratch_types=[
            pltpu.VMEM((392, 128), jnp.int32),
            pltpu.VMEM((392, 128), jnp.int32),
            pltpu.VMEM((128, FH), jnp.float32),
            pltpu.VMEM((128, FH), jnp.float32),
            pltpu.VMEM((128, FH), jnp.float32),
            pltpu.SemaphoreType.DMA,
            pltpu.SemaphoreType.DMA,
            pltpu.VMEM_SHARED((NP, FH), jnp.float32),
        ],
    )
    def _agg_kernel(*refs):
        gs = refs[0:8]
        pk_hbm = refs[8]
        os_ = refs[9:25]
        src_v, dst_v, rows0, rows1, zbuf, sem0, sem1, acc_sh = refs[25:]
        c = lax.axis_index("c")
        s = lax.axis_index("s")
        base = s * STRIPE
        _fill(zbuf, 128, FH, 0.0)
        pltpu.sync_copy(pk_hbm.at[s], src_v)

        mask = jnp.full((16,), 0xFFFF, jnp.int32)

        def unpack(j, carry):
            for k in range(8):
                v = src_v[j, k * 16:(k + 1) * 16]
                dst_v[j, k * 16:(k + 1) * 16] = lax.shift_right_logical(v, 16)
                src_v[j, k * 16:(k + 1) * 16] = lax.bitwise_and(v, mask) * 2
            return carry

        lax.fori_loop(0, 392, unpack, 0)

        def run(cid):
            for p in range(4):
                k = cid * 4 + p
                _agg_chunk(gs[k], os_[2 * k], src_v, dst_v, rows0, rows1,
                           sem0, sem1, zbuf, acc_sh, base)
                _bump(src_v, 1)
                _agg_chunk(gs[k], os_[2 * k + 1], src_v, dst_v, rows0, rows1,
                           sem0, sem1, zbuf, acc_sh, base)
                if p != 3:
                    _bump(src_v, -1)

        @pl.when(c == 0)
        def _():
            run(0)

        @pl.when(c == 1)
        def _():
            run(1)

    return _deg_kernel, _agg_kernel


def _dinv_of(degp_ref):
    deg = degp_ref[0, :, 0:1] + degp_ref[1, :, 0:1] + 1.0
    return lax.rsqrt(deg)


def _tc1_body(x_ref, w1_ref, degp_ref, *gouts):
    dinv = _dinv_of(degp_ref)
    h = jnp.dot(x_ref[...], w1_ref[...], preferred_element_type=jnp.float32)
    g = h * dinv
    for i, r in enumerate(gouts):
        r[...] = g[:, i * FC:(i + 1) * FC]


def _tc2_body(*refs):
    srefs, grefs = refs[0:16], refs[16:24]
    degp_ref, b1_ref, w2_ref = refs[24:27]
    gouts = refs[27:35]
    dinv = _dinv_of(degp_ref)
    S = jnp.concatenate([r[...] for r in srefs], axis=1)
    G = jnp.concatenate([r[...] for r in grefs], axis=1)
    h = jnp.maximum(dinv * (S + G) + b1_ref[...], 0.0)
    g = jnp.dot(h, w2_ref[...], preferred_element_type=jnp.float32) * dinv
    for i, r in enumerate(gouts):
        r[...] = g[:, i * FC:(i + 1) * FC]


def _tc3_body(*refs):
    srefs, grefs = refs[0:16], refs[16:24]
    degp_ref, b2_ref, w3_ref, b3_ref = refs[24:28]
    out_ref = refs[28]
    dinv = _dinv_of(degp_ref)
    S = jnp.concatenate([r[...] for r in srefs], axis=1)
    G = jnp.concatenate([r[...] for r in grefs], axis=1)
    h = jnp.maximum(dinv * (S + G) + b2_ref[...], 0.0)
    logits = jnp.dot(h, w3_ref[...],
                     preferred_element_type=jnp.float32) + b3_ref[...]
    m = jnp.max(logits, axis=1, keepdims=True)
    e = jnp.exp(logits - m)
    p = e / jnp.sum(e, axis=1, keepdims=True)
    out_ref[...] = p[:, :5]


_row_spec = pl.BlockSpec((BN, HID), lambda i: (i, 0))
_chunk_spec = pl.BlockSpec((BN, FC), lambda i: (i, 0))
_half_spec = pl.BlockSpec((BN, FH), lambda i: (i, 0))
_deg_spec = pl.BlockSpec((2, BN, FH), lambda i: (0, i, 0))
_w_spec = pl.BlockSpec((HID, HID), lambda i: (0, 0))
_b_spec = pl.BlockSpec((1, HID), lambda i: (0, 0))

_GRID = (NP // BN,)


@functools.cache
def _build_tc_kernels(interpret=False):
    chunks_out = tuple(jax.ShapeDtypeStruct((NP, FC), jnp.float32)
                       for _ in range(8))
    tc1 = pl.pallas_call(
        _tc1_body,
        grid=_GRID,
        in_specs=[_row_spec, _w_spec, _deg_spec],
        out_specs=tuple(_chunk_spec for _ in range(8)),
        out_shape=chunks_out,
        interpret=interpret,
    )
    tc2 = pl.pallas_call(
        _tc2_body,
        grid=_GRID,
        in_specs=[_half_spec] * 16 + [_chunk_spec] * 8 + [_deg_spec, _b_spec,
                                                          _w_spec],
        out_specs=tuple(_chunk_spec for _ in range(8)),
        out_shape=chunks_out,
        interpret=interpret,
    )
    tc3 = pl.pallas_call(
        _tc3_body,
        grid=_GRID,
        in_specs=[_half_spec] * 16 + [_chunk_spec] * 8 + [_deg_spec, _b_spec,
                                                          _w_spec, _b_spec],
        out_specs=pl.BlockSpec((BN, 5), lambda i: (i, 0)),
        out_shape=jax.ShapeDtypeStruct((NP, 5), jnp.float32),
        interpret=interpret,
    )
    return tc1, tc2, tc3


def kernel(x, edge_index, batch, W1, b1, W2, b2, W3, b3):
    x0 = x[:, -1, :]
    xp = jnp.pad(x0, ((0, NP - N), (0, HID - x0.shape[1])))
    W1p = jnp.pad(W1, ((0, HID - W1.shape[0]), (0, 0)))
    W3p = jnp.pad(W3, ((0, 0), (0, HID - W3.shape[1])))
    b3p = jnp.concatenate([b3, jnp.full((HID - b3.shape[0],), -1e30,
                                        jnp.float32)])

    src = edge_index[0]
    dst = edge_index[1]
    j = jnp.arange(EP - E, dtype=jnp.int32)
    pad_src = N + (j % (NP - N))
    pad_dst = N + ((j * 7 + 3) % (NP - N))
    srcp = jnp.concatenate([src, pad_src])
    dstp = jnp.concatenate([dst, pad_dst])
    pk = lax.bitcast_convert_type(
        (dstp.astype(jnp.uint32) << jnp.uint32(16)) | srcp.astype(jnp.uint32),
        jnp.int32)
    pk16 = pk.reshape(16, 392, 128)
    dst32 = dstp.reshape(32, 196, 128)

    deg_k, agg_k = _build_sc_kernels()
    tc1, tc2, tc3 = _build_tc_kernels()
    degp = deg_k(dst32).reshape(2, NP, FH)

    g1 = tc1(xp, W1p, degp)
    s1 = agg_k(*(g.reshape(2 * NP, FH) for g in g1), pk16)
    g2 = tc2(*s1, *g1, degp, b1.reshape(1, HID), W2)
    s2 = agg_k(*(g.reshape(2 * NP, FH) for g in g2), pk16)
    out = tc3(*s2, *g2, degp, b2.reshape(1, HID), W3p, b3p.reshape(1, HID))
    return out[:N]


# speedup vs baseline: 1.0001x; 1.0001x over previous
import functools

import jax
import jax.numpy as jnp
from jax import lax
from jax.experimental import pallas as pl
from jax.experimental.pallas import tpu as pltpu
from jax.experimental.pallas import tpu_sc as plsc

N = 50000
NP = 50176
E = 800000
EP = 802816
HID = 128
FC = 16
FH = 8
BN = 512
STRIPE = NP // 16


def _fill(ref, rows, w, val):
    v = jnp.full((16,), val, ref.dtype)

    def body(i, carry):
        for k in range(w // 16):
            ref[i, k * 16:(k + 1) * 16] = v
        return carry

    lax.fori_loop(0, rows, body, 0)


def _zero_stripe(zbuf, acc_sh, base):
    def body(k, carry):
        pltpu.sync_copy(zbuf, acc_sh.at[pl.ds(base + k * 128, 128)])
        return carry

    lax.fori_loop(0, STRIPE // 128, body, 0)
    rem = STRIPE % 128
    if rem:
        pltpu.sync_copy(zbuf.at[pl.ds(0, rem)],
                        acc_sh.at[pl.ds(base + (STRIPE // 128) * 128, rem)])


def _agg_chunk(g2, out, src_v, dst_v, rows0, rows1, sem0, sem1, zbuf,
               acc_sh, base):
    _zero_stripe(zbuf, acc_sh, base)
    plsc.subcore_barrier()

    def body(u, carry):
        j0 = 2 * u
        j1 = 2 * u + 1
        h0 = pltpu.async_copy(g2.at[src_v.at[j0]], rows0, sem0)
        h1 = pltpu.async_copy(g2.at[src_v.at[j1]], rows1, sem1)
        h0.wait()
        pltpu.sync_copy(rows0, acc_sh.at[dst_v.at[j0]], add=True)
        h1.wait()
        pltpu.sync_copy(rows1, acc_sh.at[dst_v.at[j1]], add=True)
        return carry

    lax.fori_loop(0, 196, body, 0)
    plsc.subcore_barrier()
    pltpu.sync_copy(acc_sh.at[pl.ds(base, STRIPE)], out.at[pl.ds(base, STRIPE)])
    plsc.subcore_barrier()


def _bump(src_v, delta):
    v = jnp.full((16,), delta, jnp.int32)

    def body(j, carry):
        for k in range(8):
            src_v[j, k * 16:(k + 1) * 16] = src_v[j, k * 16:(k + 1) * 16] + v
        return carry

    lax.fori_loop(0, 392, body, 0)


@functools.cache
def _build_sc_kernels():
    mesh = plsc.VectorSubcoreMesh(core_axis_name="c", subcore_axis_name="s")

    @functools.partial(
        pl.kernel,
        mesh=mesh,
        compiler_params=pltpu.CompilerParams(use_tc_tiling_on_sc=False),
        out_type=jax.ShapeDtypeStruct((2 * NP, FH), jnp.float32),
        scratch_types=[
            pltpu.VMEM((196, 128), jnp.int32),
            pltpu.VMEM((128, FH), jnp.float32),
            pltpu.VMEM((128, FH), jnp.float32),
            pltpu.VMEM_SHARED((NP, FH), jnp.float32),
        ],
    )
    def _deg_kernel(dst_hbm, out_hbm, idx_v, ones_v, zbuf, acc_sh):
        c = lax.axis_index("c")
        s = lax.axis_index("s")
        w = c * 16 + s
        base = s * STRIPE
        _fill(ones_v, 128, FH, 1.0)
        _fill(zbuf, 128, FH, 0.0)
        _zero_stripe(zbuf, acc_sh, base)
        pltpu.sync_copy(dst_hbm.at[w], idx_v)
        plsc.subcore_barrier()

        def body(j, carry):
            pltpu.sync_copy(ones_v, acc_sh.at[idx_v.at[j]], add=True)
            return carry

        lax.fori_loop(0, 196, body, 0)
        plsc.subcore_barrier()
        pltpu.sync_copy(acc_sh.at[pl.ds(base, STRIPE)],
                        out_hbm.at[pl.ds(c * NP + base, STRIPE)])

    @functools.partial(
        pl.kernel,
        mesh=mesh,
        compiler_params=pltpu.CompilerParams(
            use_tc_tiling_on_sc=False,
            disable_bounds_checks=True,
            disable_semaphore_checks=True),
        out_type=tuple(jax.ShapeDtypeStruct((NP, FH), jnp.float32)
                       for _ in range(16)),
        scratch_types=[
            pltpu.VMEM((392, 128), jnp.int32),
            pltpu.VMEM((392, 128), jnp.int32),
            pltpu.VMEM((128, FH), jnp.float32),
            pltpu.VMEM((128, FH), jnp.float32),
            pltpu.VMEM((128, FH), jnp.float32),
            pltpu.SemaphoreType.DMA,
            pltpu.SemaphoreType.DMA,
            pltpu.VMEM_SHARED((NP, FH), jnp.float32),
        ],
    )
    def _agg_kernel(*refs):
        gs = refs[0:8]
        pk_hbm = refs[8]
        os_ = refs[9:25]
        src_v, dst_v, rows0, rows1, zbuf, sem0, sem1, acc_sh = refs[25:]
        c = lax.axis_index("c")
        s = lax.axis_index("s")
        base = s * STRIPE
        _fill(zbuf, 128, FH, 0.0)
        pltpu.sync_copy(pk_hbm.at[s], src_v)

        mask = jnp.full((16,), 0xFFFF, jnp.int32)

        def unpack(j, carry):
            for k in range(8):
                v = src_v[j, k * 16:(k + 1) * 16]
                dst_v[j, k * 16:(k + 1) * 16] = lax.shift_right_logical(v, 16)
                src_v[j, k * 16:(k + 1) * 16] = lax.bitwise_and(v, mask) * 2
            return carry

        lax.fori_loop(0, 392, unpack, 0)

        def run(cid):
            for p in range(4):
                k = cid * 4 + p
                _agg_chunk(gs[k], os_[2 * k], src_v, dst_v, rows0, rows1,
                           sem0, sem1, zbuf, acc_sh, base)
                _bump(src_v, 1)
                _agg_chunk(gs[k], os_[2 * k + 1], src_v, dst_v, rows0, rows1,
                           sem0, sem1, zbuf, acc_sh, base)
                if p != 3:
                    _bump(src_v, -1)

        @pl.when(c == 0)
        def _():
            run(0)

        @pl.when(c == 1)
        def _():
            run(1)

    return _deg_kernel, _agg_kernel


def _dinv_of(degp_ref):
    deg = degp_ref[0, :, 0:1] + degp_ref[1, :, 0:1] + 1.0
    return lax.rsqrt(deg)


def _tc1_body(x_ref, w1_ref, degp_ref, *gouts):
    dinv = _dinv_of(degp_ref)
    h = jnp.dot(x_ref[...], w1_ref[...], preferred_element_type=jnp.float32)
    g = h * dinv
    for i, r in enumerate(gouts):
        r[...] = g[:, i * FC:(i + 1) * FC]


def _tc2_body(*refs):
    srefs, grefs = refs[0:16], refs[16:24]
    degp_ref, b1_ref, w2_ref = refs[24:27]
    gouts = refs[27:35]
    dinv = _dinv_of(degp_ref)
    S = jnp.concatenate([r[...] for r in srefs], axis=1)
    G = jnp.concatenate([r[...] for r in grefs], axis=1)
    h = jnp.maximum(dinv * (S + G) + b1_ref[...], 0.0)
    g = jnp.dot(h, w2_ref[...], preferred_element_type=jnp.float32) * dinv
    for i, r in enumerate(gouts):
        r[...] = g[:, i * FC:(i + 1) * FC]


def _tc3_body(*refs):
    srefs, grefs = refs[0:16], refs[16:24]
    degp_ref, b2_ref, w3_ref, b3_ref = refs[24:28]
    out_ref = refs[28]
    dinv = _dinv_of(degp_ref)
    S = jnp.concatenate([r[...] for r in srefs], axis=1)
    G = jnp.concatenate([r[...] for r in grefs], axis=1)
    h = jnp.maximum(dinv * (S + G) + b2_ref[...], 0.0)
    logits = jnp.dot(h, w3_ref[...],
                     preferred_element_type=jnp.float32) + b3_ref[...]
    m = jnp.max(logits, axis=1, keepdims=True)
    e = jnp.exp(logits - m)
    p = e / jnp.sum(e, axis=1, keepdims=True)
    out_ref[...] = p[:, :5]


_row_spec = pl.BlockSpec((BN, HID), lambda i: (i, 0))
_chunk_spec = pl.BlockSpec((BN, FC), lambda i: (i, 0))
_half_spec = pl.BlockSpec((BN, FH), lambda i: (i, 0))
_deg_spec = pl.BlockSpec((2, BN, FH), lambda i: (0, i, 0))
_w_spec = pl.BlockSpec((HID, HID), lambda i: (0, 0))
_b_spec = pl.BlockSpec((1, HID), lambda i: (0, 0))

_GRID = (NP // BN,)


@functools.cache
def _build_tc_kernels(interpret=False):
    chunks_out = tuple(jax.ShapeDtypeStruct((NP, FC), jnp.float32)
                       for _ in range(8))
    tc1 = pl.pallas_call(
        _tc1_body,
        grid=_GRID,
        in_specs=[_row_spec, _w_spec, _deg_spec],
        out_specs=tuple(_chunk_spec for _ in range(8)),
        out_shape=chunks_out,
        interpret=interpret,
    )
    tc2 = pl.pallas_call(
        _tc2_body,
        grid=_GRID,
        in_specs=[_half_spec] * 16 + [_chunk_spec] * 8 + [_deg_spec, _b_spec,
                                                          _w_spec],
        out_specs=tuple(_chunk_spec for _ in range(8)),
        out_shape=chunks_out,
        interpret=interpret,
    )
    tc3 = pl.pallas_call(
        _tc3_body,
        grid=_GRID,
        in_specs=[_half_spec] * 16 + [_chunk_spec] * 8 + [_deg_spec, _b_spec,
                                                          _w_spec, _b_spec],
        out_specs=pl.BlockSpec((BN, 5), lambda i: (i, 0)),
        out_shape=jax.ShapeDtypeStruct((NP, 5), jnp.float32),
        interpret=interpret,
    )
    return tc1, tc2, tc3


def kernel(x, edge_index, batch, W1, b1, W2, b2, W3, b3):
    x0 = x[:, -1, :]
    xp = jnp.pad(x0, ((0, NP - N), (0, HID - x0.shape[1])))
    W1p = jnp.pad(W1, ((0, HID - W1.shape[0]), (0, 0)))
    W3p = jnp.pad(W3, ((0, 0), (0, HID - W3.shape[1])))
    b3p = jnp.concatenate([b3, jnp.full((HID - b3.shape[0],), -1e30,
                                        jnp.float32)])

    src = edge_index[0]
    dst = edge_index[1]
    j = jnp.arange(EP - E, dtype=jnp.int32)
    pad_src = N + (j % (NP - N))
    pad_dst = N + ((j * 7 + 3) % (NP - N))
    srcp = jnp.concatenate([src, pad_src])
    dstp = jnp.concatenate([dst, pad_dst])
    pk = lax.bitcast_convert_type(
        (dstp.astype(jnp.uint32) << jnp.uint32(16)) | srcp.astype(jnp.uint32),
        jnp.int32)
    pk16 = pk.reshape(16, 392, 128)
    dst32 = dstp.reshape(32, 196, 128)

    deg_k, agg_k = _build_sc_kernels()
    tc1, tc2, tc3 = _build_tc_kernels()
    degp = deg_k(dst32).reshape(2, NP, FH)

    g1 = tc1(xp, W1p, degp)
    s1 = agg_k(*(g.reshape(2 * NP, FH) for g in g1), pk16)
    g2 = tc2(*s1, *g1, degp, b1.reshape(1, HID), W2)
    s2 = agg_k(*(g.reshape(2 * NP, FH) for g in g2), pk16)
    out = tc3(*s2, *g2, degp, b2.reshape(1, HID), W3p, b3p.reshape(1, HID))
    return out[:N]


# speedup vs baseline: 1.3458x; 1.3457x over previous
import functools

import jax
import jax.numpy as jnp
from jax import lax
from jax.experimental import pallas as pl
from jax.experimental.pallas import tpu as pltpu
from jax.experimental.pallas import tpu_sc as plsc

N = 50000
NP = 50176
E = 800000
EP = 802816
HID = 128
FC = 16
FH = 8
BN = 512
STRIPE = NP // 16


def _fill(ref, rows, w, val):
    v = jnp.full((16,), val, ref.dtype)

    def body(i, carry):
        for k in range(w // 16):
            ref[i, k * 16:(k + 1) * 16] = v
        return carry

    lax.fori_loop(0, rows, body, 0)


def _zero_stripe(zbuf, acc_sh, base):
    def body(k, carry):
        pltpu.sync_copy(zbuf, acc_sh.at[pl.ds(base + k * 128, 128)])
        return carry

    lax.fori_loop(0, STRIPE // 128, body, 0)
    rem = STRIPE % 128
    if rem:
        pltpu.sync_copy(zbuf.at[pl.ds(0, rem)],
                        acc_sh.at[pl.ds(base + (STRIPE // 128) * 128, rem)])


def _agg_chunk(g2, out, src_v, dst_v, rows, gsem, ssem, zbuf,
               acc_sh, base):
    _zero_stripe(zbuf, acc_sh, base)
    plsc.subcore_barrier()

    def body(u, carry):
        hs = []
        for v in range(4):
            hs.append(pltpu.async_copy(g2.at[src_v.at[4 * u + v]],
                                       rows[v], gsem[v]))
        ss = []
        for v in range(4):
            hs[v].wait()
            ss.append(pltpu.async_copy(rows[v], acc_sh.at[dst_v.at[4 * u + v]],
                                       ssem[v], add=True))
        for v in range(4):
            ss[v].wait()
        return carry

    lax.fori_loop(0, 98, body, 0)
    plsc.subcore_barrier()
    pltpu.sync_copy(acc_sh.at[pl.ds(base, STRIPE)], out.at[pl.ds(base, STRIPE)])
    plsc.subcore_barrier()


def _bump(src_v, delta):
    v = jnp.full((16,), delta, jnp.int32)

    def body(j, carry):
        for k in range(8):
            src_v[j, k * 16:(k + 1) * 16] = src_v[j, k * 16:(k + 1) * 16] + v
        return carry

    lax.fori_loop(0, 392, body, 0)


@functools.cache
def _build_sc_kernels():
    mesh = plsc.VectorSubcoreMesh(core_axis_name="c", subcore_axis_name="s")

    @functools.partial(
        pl.kernel,
        mesh=mesh,
        compiler_params=pltpu.CompilerParams(use_tc_tiling_on_sc=False),
        out_type=jax.ShapeDtypeStruct((2 * NP, FH), jnp.float32),
        scratch_types=[
            pltpu.VMEM((196, 128), jnp.int32),
            pltpu.VMEM((128, FH), jnp.float32),
            pltpu.VMEM((128, FH), jnp.float32),
            pltpu.VMEM_SHARED((NP, FH), jnp.float32),
        ],
    )
    def _deg_kernel(dst_hbm, out_hbm, idx_v, ones_v, zbuf, acc_sh):
        c = lax.axis_index("c")
        s = lax.axis_index("s")
        w = c * 16 + s
        base = s * STRIPE
        _fill(ones_v, 128, FH, 1.0)
        _fill(zbuf, 128, FH, 0.0)
        _zero_stripe(zbuf, acc_sh, base)
        pltpu.sync_copy(dst_hbm.at[w], idx_v)
        plsc.subcore_barrier()

        def body(j, carry):
            pltpu.sync_copy(ones_v, acc_sh.at[idx_v.at[j]], add=True)
            return carry

        lax.fori_loop(0, 196, body, 0)
        plsc.subcore_barrier()
        pltpu.sync_copy(acc_sh.at[pl.ds(base, STRIPE)],
                        out_hbm.at[pl.ds(c * NP + base, STRIPE)])

    @functools.partial(
        pl.kernel,
        mesh=mesh,
        compiler_params=pltpu.CompilerParams(
            use_tc_tiling_on_sc=False,
            disable_bounds_checks=True,
            disable_semaphore_checks=True),
        out_type=tuple(jax.ShapeDtypeStruct((NP, FH), jnp.float32)
                       for _ in range(16)),
        scratch_types=[
            pltpu.VMEM((392, 128), jnp.int32),
            pltpu.VMEM((392, 128), jnp.int32),
            pltpu.VMEM((128, FH), jnp.float32),
            pltpu.VMEM((128, FH), jnp.float32),
            pltpu.VMEM((128, FH), jnp.float32),
            pltpu.VMEM((128, FH), jnp.float32),
            pltpu.VMEM((128, FH), jnp.float32),
            pltpu.SemaphoreType.DMA,
            pltpu.SemaphoreType.DMA,
            pltpu.SemaphoreType.DMA,
            pltpu.SemaphoreType.DMA,
            pltpu.SemaphoreType.DMA,
            pltpu.SemaphoreType.DMA,
            pltpu.SemaphoreType.DMA,
            pltpu.SemaphoreType.DMA,
            pltpu.VMEM_SHARED((NP, FH), jnp.float32),
        ],
    )
    def _agg_kernel(*refs):
        gs = refs[0:8]
        pk_hbm = refs[8]
        os_ = refs[9:25]
        src_v = refs[25]
        dst_v = refs[26]
        rows = refs[27:31]
        zbuf = refs[31]
        gsem = refs[32:36]
        ssem = refs[36:40]
        acc_sh = refs[40]
        c = lax.axis_index("c")
        s = lax.axis_index("s")
        base = s * STRIPE
        _fill(zbuf, 128, FH, 0.0)
        pltpu.sync_copy(pk_hbm.at[s], src_v)

        mask = jnp.full((16,), 0xFFFF, jnp.int32)

        def unpack(j, carry):
            for k in range(8):
                v = src_v[j, k * 16:(k + 1) * 16]
                dst_v[j, k * 16:(k + 1) * 16] = lax.shift_right_logical(v, 16)
                src_v[j, k * 16:(k + 1) * 16] = lax.bitwise_and(v, mask) * 2
            return carry

        lax.fori_loop(0, 392, unpack, 0)

        def run(cid):
            for p in range(4):
                k = cid * 4 + p
                _agg_chunk(gs[k], os_[2 * k], src_v, dst_v, rows,
                           gsem, ssem, zbuf, acc_sh, base)
                _bump(src_v, 1)
                _agg_chunk(gs[k], os_[2 * k + 1], src_v, dst_v, rows,
                           gsem, ssem, zbuf, acc_sh, base)
                if p != 3:
                    _bump(src_v, -1)

        @pl.when(c == 0)
        def _():
            run(0)

        @pl.when(c == 1)
        def _():
            run(1)

    return _deg_kernel, _agg_kernel


def _dinv_of(degp_ref):
    deg = degp_ref[0, :, 0:1] + degp_ref[1, :, 0:1] + 1.0
    return lax.rsqrt(deg)


def _tc1_body(x_ref, w1_ref, degp_ref, *gouts):
    dinv = _dinv_of(degp_ref)
    h = jnp.dot(x_ref[...], w1_ref[...], preferred_element_type=jnp.float32)
    g = h * dinv
    for i, r in enumerate(gouts):
        r[...] = g[:, i * FC:(i + 1) * FC]


def _tc2_body(*refs):
    srefs, grefs = refs[0:16], refs[16:24]
    degp_ref, b1_ref, w2_ref = refs[24:27]
    gouts = refs[27:35]
    dinv = _dinv_of(degp_ref)
    S = jnp.concatenate([r[...] for r in srefs], axis=1)
    G = jnp.concatenate([r[...] for r in grefs], axis=1)
    h = jnp.maximum(dinv * (S + G) + b1_ref[...], 0.0)
    g = jnp.dot(h, w2_ref[...], preferred_element_type=jnp.float32) * dinv
    for i, r in enumerate(gouts):
        r[...] = g[:, i * FC:(i + 1) * FC]


def _tc3_body(*refs):
    srefs, grefs = refs[0:16], refs[16:24]
    degp_ref, b2_ref, w3_ref, b3_ref = refs[24:28]
    out_ref = refs[28]
    dinv = _dinv_of(degp_ref)
    S = jnp.concatenate([r[...] for r in srefs], axis=1)
    G = jnp.concatenate([r[...] for r in grefs], axis=1)
    h = jnp.maximum(dinv * (S + G) + b2_ref[...], 0.0)
    logits = jnp.dot(h, w3_ref[...],
                     preferred_element_type=jnp.float32) + b3_ref[...]
    m = jnp.max(logits, axis=1, keepdims=True)
    e = jnp.exp(logits - m)
    p = e / jnp.sum(e, axis=1, keepdims=True)
    out_ref[...] = p[:, :5]


_row_spec = pl.BlockSpec((BN, HID), lambda i: (i, 0))
_chunk_spec = pl.BlockSpec((BN, FC), lambda i: (i, 0))
_half_spec = pl.BlockSpec((BN, FH), lambda i: (i, 0))
_deg_spec = pl.BlockSpec((2, BN, FH), lambda i: (0, i, 0))
_w_spec = pl.BlockSpec((HID, HID), lambda i: (0, 0))
_b_spec = pl.BlockSpec((1, HID), lambda i: (0, 0))

_GRID = (NP // BN,)


@functools.cache
def _build_tc_kernels(interpret=False):
    chunks_out = tuple(jax.ShapeDtypeStruct((NP, FC), jnp.float32)
                       for _ in range(8))
    tc1 = pl.pallas_call(
        _tc1_body,
        grid=_GRID,
        in_specs=[_row_spec, _w_spec, _deg_spec],
        out_specs=tuple(_chunk_spec for _ in range(8)),
        out_shape=chunks_out,
        interpret=interpret,
    )
    tc2 = pl.pallas_call(
        _tc2_body,
        grid=_GRID,
        in_specs=[_half_spec] * 16 + [_chunk_spec] * 8 + [_deg_spec, _b_spec,
                                                          _w_spec],
        out_specs=tuple(_chunk_spec for _ in range(8)),
        out_shape=chunks_out,
        interpret=interpret,
    )
    tc3 = pl.pallas_call(
        _tc3_body,
        grid=_GRID,
        in_specs=[_half_spec] * 16 + [_chunk_spec] * 8 + [_deg_spec, _b_spec,
                                                          _w_spec, _b_spec],
        out_specs=pl.BlockSpec((BN, 5), lambda i: (i, 0)),
        out_shape=jax.ShapeDtypeStruct((NP, 5), jnp.float32),
        interpret=interpret,
    )
    return tc1, tc2, tc3


def kernel(x, edge_index, batch, W1, b1, W2, b2, W3, b3):
    x0 = x[:, -1, :]
    xp = jnp.pad(x0, ((0, NP - N), (0, HID - x0.shape[1])))
    W1p = jnp.pad(W1, ((0, HID - W1.shape[0]), (0, 0)))
    W3p = jnp.pad(W3, ((0, 0), (0, HID - W3.shape[1])))
    b3p = jnp.concatenate([b3, jnp.full((HID - b3.shape[0],), -1e30,
                                        jnp.float32)])

    src = edge_index[0]
    dst = edge_index[1]
    j = jnp.arange(EP - E, dtype=jnp.int32)
    pad_src = N + (j % (NP - N))
    pad_dst = N + ((j * 7 + 3) % (NP - N))
    srcp = jnp.concatenate([src, pad_src])
    dstp = jnp.concatenate([dst, pad_dst])
    pk = lax.bitcast_convert_type(
        (dstp.astype(jnp.uint32) << jnp.uint32(16)) | srcp.astype(jnp.uint32),
        jnp.int32)
    pk16 = pk.reshape(16, 392, 128)
    dst32 = dstp.reshape(32, 196, 128)

    deg_k, agg_k = _build_sc_kernels()
    tc1, tc2, tc3 = _build_tc_kernels()
    degp = deg_k(dst32).reshape(2, NP, FH)

    g1 = tc1(xp, W1p, degp)
    s1 = agg_k(*(g.reshape(2 * NP, FH) for g in g1), pk16)
    g2 = tc2(*s1, *g1, degp, b1.reshape(1, HID), W2)
    s2 = agg_k(*(g.reshape(2 * NP, FH) for g in g2), pk16)
    out = tc3(*s2, *g2, degp, b2.reshape(1, HID), W3p, b3p.reshape(1, HID))
    return out[:N]


# speedup vs baseline: 1.3723x; 1.0197x over previous
import functools

import jax
import jax.numpy as jnp
from jax import lax
from jax.experimental import pallas as pl
from jax.experimental.pallas import tpu as pltpu
from jax.experimental.pallas import tpu_sc as plsc

N = 50000
NP = 50176
E = 800000
EP = 802816
HID = 128
FC = 16
FH = 8
BN = 1024
STRIPE = NP // 16


def _fill(ref, rows, w, val):
    v = jnp.full((16,), val, ref.dtype)

    def body(i, carry):
        for k in range(w // 16):
            ref[i, k * 16:(k + 1) * 16] = v
        return carry

    lax.fori_loop(0, rows, body, 0)


def _zero_stripe(zbuf, acc_sh, base):
    def body(k, carry):
        pltpu.sync_copy(zbuf, acc_sh.at[pl.ds(base + k * 128, 128)])
        return carry

    lax.fori_loop(0, STRIPE // 128, body, 0)
    rem = STRIPE % 128
    if rem:
        pltpu.sync_copy(zbuf.at[pl.ds(0, rem)],
                        acc_sh.at[pl.ds(base + (STRIPE // 128) * 128, rem)])


def _agg_chunk(g2, out, src_v, dst_v, rows, gsem, ssem, zbuf,
               acc_sh, base):
    _zero_stripe(zbuf, acc_sh, base)
    plsc.subcore_barrier()

    def body(u, carry):
        hs = []
        for v in range(4):
            hs.append(pltpu.async_copy(g2.at[src_v.at[4 * u + v]],
                                       rows[v], gsem[v]))
        ss = []
        for v in range(4):
            hs[v].wait()
            ss.append(pltpu.async_copy(rows[v], acc_sh.at[dst_v.at[4 * u + v]],
                                       ssem[v], add=True))
        for v in range(4):
            ss[v].wait()
        return carry

    lax.fori_loop(0, 98, body, 0)
    plsc.subcore_barrier()
    pltpu.sync_copy(acc_sh.at[pl.ds(base, STRIPE)], out.at[pl.ds(base, STRIPE)])
    plsc.subcore_barrier()


def _bump(src_v, delta):
    v = jnp.full((16,), delta, jnp.int32)

    def body(j, carry):
        for k in range(8):
            src_v[j, k * 16:(k + 1) * 16] = src_v[j, k * 16:(k + 1) * 16] + v
        return carry

    lax.fori_loop(0, 392, body, 0)


@functools.cache
def _build_sc_kernels():
    mesh = plsc.VectorSubcoreMesh(core_axis_name="c", subcore_axis_name="s")

    @functools.partial(
        pl.kernel,
        mesh=mesh,
        compiler_params=pltpu.CompilerParams(use_tc_tiling_on_sc=False),
        out_type=jax.ShapeDtypeStruct((2 * NP, FH), jnp.float32),
        scratch_types=[
            pltpu.VMEM((196, 128), jnp.int32),
            pltpu.VMEM((128, FH), jnp.float32),
            pltpu.VMEM((128, FH), jnp.float32),
            pltpu.VMEM_SHARED((NP, FH), jnp.float32),
        ],
    )
    def _deg_kernel(dst_hbm, out_hbm, idx_v, ones_v, zbuf, acc_sh):
        c = lax.axis_index("c")
        s = lax.axis_index("s")
        w = c * 16 + s
        base = s * STRIPE
        _fill(ones_v, 128, FH, 1.0)
        _fill(zbuf, 128, FH, 0.0)
        _zero_stripe(zbuf, acc_sh, base)
        pltpu.sync_copy(dst_hbm.at[w], idx_v)
        plsc.subcore_barrier()

        def body(j, carry):
            pltpu.sync_copy(ones_v, acc_sh.at[idx_v.at[j]], add=True)
            return carry

        lax.fori_loop(0, 196, body, 0)
        plsc.subcore_barrier()
        pltpu.sync_copy(acc_sh.at[pl.ds(base, STRIPE)],
                        out_hbm.at[pl.ds(c * NP + base, STRIPE)])

    @functools.partial(
        pl.kernel,
        mesh=mesh,
        compiler_params=pltpu.CompilerParams(
            use_tc_tiling_on_sc=False,
            disable_bounds_checks=True,
            disable_semaphore_checks=True),
        out_type=tuple(jax.ShapeDtypeStruct((NP, FH), jnp.float32)
                       for _ in range(16)),
        scratch_types=(
            [pltpu.VMEM((392, 128), jnp.int32)] * 2
            + [pltpu.VMEM((128, FH), jnp.float32)] * 5
            + [pltpu.SemaphoreType.DMA] * 8
            + [pltpu.VMEM_SHARED((NP, FH), jnp.float32)]
        ),
    )
    def _agg_kernel(*refs):
        gs = refs[0:8]
        pk_hbm = refs[8]
        os_ = refs[9:25]
        src_v = refs[25]
        dst_v = refs[26]
        rows = refs[27:31]
        zbuf = refs[31]
        gsem = refs[32:36]
        ssem = refs[36:40]
        acc_sh = refs[40]
        c = lax.axis_index("c")
        s = lax.axis_index("s")
        base = s * STRIPE
        _fill(zbuf, 128, FH, 0.0)
        pltpu.sync_copy(pk_hbm.at[s], src_v)

        mask = jnp.full((16,), 0xFFFF, jnp.int32)

        def unpack(j, carry):
            for k in range(8):
                v = src_v[j, k * 16:(k + 1) * 16]
                dst_v[j, k * 16:(k + 1) * 16] = lax.shift_right_logical(v, 16)
                src_v[j, k * 16:(k + 1) * 16] = lax.bitwise_and(v, mask) * 2
            return carry

        lax.fori_loop(0, 392, unpack, 0)

        def run(cid):
            for p in range(4):
                k = cid * 4 + p
                _agg_chunk(gs[k], os_[2 * k], src_v, dst_v, rows,
                           gsem, ssem, zbuf, acc_sh, base)
                _bump(src_v, 1)
                _agg_chunk(gs[k], os_[2 * k + 1], src_v, dst_v, rows,
                           gsem, ssem, zbuf, acc_sh, base)
                if p != 3:
                    _bump(src_v, -1)

        @pl.when(c == 0)
        def _():
            run(0)

        @pl.when(c == 1)
        def _():
            run(1)

    return _deg_kernel, _agg_kernel


def _dinv_of(degp_ref):
    deg = degp_ref[0, :, 0:1] + degp_ref[1, :, 0:1] + 1.0
    return lax.rsqrt(deg)


def _tc1_body(x_ref, w1_ref, degp_ref, *gouts):
    dinv = _dinv_of(degp_ref)
    h = jnp.dot(x_ref[...], w1_ref[...], preferred_element_type=jnp.float32)
    g = h * dinv
    for i, r in enumerate(gouts):
        r[...] = g[:, i * FC:(i + 1) * FC]


def _tc2_body(*refs):
    srefs, grefs = refs[0:16], refs[16:24]
    degp_ref, b1_ref, w2_ref = refs[24:27]
    gouts = refs[27:35]
    dinv = _dinv_of(degp_ref)
    S = jnp.concatenate([r[...] for r in srefs], axis=1)
    G = jnp.concatenate([r[...] for r in grefs], axis=1)
    h = jnp.maximum(dinv * (S + G) + b1_ref[...], 0.0)
    g = jnp.dot(h, w2_ref[...], preferred_element_type=jnp.float32) * dinv
    for i, r in enumerate(gouts):
        r[...] = g[:, i * FC:(i + 1) * FC]


def _tc3_body(*refs):
    srefs, grefs = refs[0:16], refs[16:24]
    degp_ref, b2_ref, w3_ref, b3_ref = refs[24:28]
    out_ref = refs[28]
    dinv = _dinv_of(degp_ref)
    S = jnp.concatenate([r[...] for r in srefs], axis=1)
    G = jnp.concatenate([r[...] for r in grefs], axis=1)
    h = jnp.maximum(dinv * (S + G) + b2_ref[...], 0.0)
    logits = jnp.dot(h, w3_ref[...],
                     preferred_element_type=jnp.float32) + b3_ref[...]
    m = jnp.max(logits, axis=1, keepdims=True)
    e = jnp.exp(logits - m)
    p = e / jnp.sum(e, axis=1, keepdims=True)
    out_ref[...] = p[:, :5]


_row_spec = pl.BlockSpec((BN, HID), lambda i: (i, 0))
_chunk_spec = pl.BlockSpec((BN, FC), lambda i: (i, 0))
_half_spec = pl.BlockSpec((BN, FH), lambda i: (i, 0))
_deg_spec = pl.BlockSpec((2, BN, FH), lambda i: (0, i, 0))
_w_spec = pl.BlockSpec((HID, HID), lambda i: (0, 0))
_b_spec = pl.BlockSpec((1, HID), lambda i: (0, 0))

_GRID = (NP // BN,)


@functools.cache
def _build_tc_kernels(interpret=False):
    chunks_out = tuple(jax.ShapeDtypeStruct((NP, FC), jnp.float32)
                       for _ in range(8))
    tc1 = pl.pallas_call(
        _tc1_body,
        grid=_GRID,
        in_specs=[_row_spec, _w_spec, _deg_spec],
        out_specs=tuple(_chunk_spec for _ in range(8)),
        out_shape=chunks_out,
        interpret=interpret,
    )
    tc2 = pl.pallas_call(
        _tc2_body,
        grid=_GRID,
        in_specs=[_half_spec] * 16 + [_chunk_spec] * 8 + [_deg_spec, _b_spec,
                                                          _w_spec],
        out_specs=tuple(_chunk_spec for _ in range(8)),
        out_shape=chunks_out,
        interpret=interpret,
    )
    tc3 = pl.pallas_call(
        _tc3_body,
        grid=_GRID,
        in_specs=[_half_spec] * 16 + [_chunk_spec] * 8 + [_deg_spec, _b_spec,
                                                          _w_spec, _b_spec],
        out_specs=pl.BlockSpec((BN, 5), lambda i: (i, 0)),
        out_shape=jax.ShapeDtypeStruct((NP, 5), jnp.float32),
        interpret=interpret,
    )
    return tc1, tc2, tc3


def kernel(x, edge_index, batch, W1, b1, W2, b2, W3, b3):
    x0 = x[:, -1, :]
    xp = jnp.pad(x0, ((0, NP - N), (0, HID - x0.shape[1])))
    W1p = jnp.pad(W1, ((0, HID - W1.shape[0]), (0, 0)))
    W3p = jnp.pad(W3, ((0, 0), (0, HID - W3.shape[1])))
    b3p = jnp.concatenate([b3, jnp.full((HID - b3.shape[0],), -1e30,
                                        jnp.float32)])

    src = edge_index[0]
    dst = edge_index[1]
    j = jnp.arange(EP - E, dtype=jnp.int32)
    pad_src = N + (j % (NP - N))
    pad_dst = N + ((j * 7 + 3) % (NP - N))
    srcp = jnp.concatenate([src, pad_src])
    dstp = jnp.concatenate([dst, pad_dst])
    pk = lax.bitcast_convert_type(
        (dstp.astype(jnp.uint32) << jnp.uint32(16)) | srcp.astype(jnp.uint32),
        jnp.int32)
    pk16 = pk.reshape(16, 392, 128)
    dst32 = dstp.reshape(32, 196, 128)

    deg_k, agg_k = _build_sc_kernels()
    tc1, tc2, tc3 = _build_tc_kernels()
    degp = deg_k(dst32).reshape(2, NP, FH)

    g1 = tc1(xp, W1p, degp)
    s1 = agg_k(*(g.reshape(2 * NP, FH) for g in g1), pk16)
    g2 = tc2(*s1, *g1, degp, b1.reshape(1, HID), W2)
    s2 = agg_k(*(g.reshape(2 * NP, FH) for g in g2), pk16)
    out = tc3(*s2, *g2, degp, b2.reshape(1, HID), W3p, b3p.reshape(1, HID))
    return out[:N]
